# Initial kernel scaffold; baseline (speedup 1.0000x reference)
import jax
import jax.numpy as jnp
from jax.experimental import pallas as pl


def kernel(ids, table):
    raise NotImplementedError("write your pallas kernel here")


# speedup vs baseline: 6.2824x; 6.2824x over previous
import functools

import jax
import jax.numpy as jnp
from jax import lax
from jax.experimental import pallas as pl
from jax.experimental.pallas import tpu as pltpu
from jax.experimental.pallas import tpu_sc as plsc


def _sc_gather(n_rows, dim, n_workers, chunk):
    n_chunks_per_w = (n_rows // n_workers) // chunk
    b_per_w = n_rows // n_workers
    mesh = plsc.VectorSubcoreMesh(core_axis_name="c", subcore_axis_name="s")

    @functools.partial(
        pl.kernel,
        out_type=jax.ShapeDtypeStruct((n_rows, dim), jnp.float32),
        mesh=mesh,
        scratch_types=[
            pltpu.VMEM((chunk,), jnp.int32),
            pltpu.VMEM((chunk, dim), jnp.float32),
            pltpu.SemaphoreType.DMA,
        ],
        compiler_params=pltpu.CompilerParams(use_tc_tiling_on_sc=False),
    )
    def k(idx_hbm, table_hbm, out_hbm, idx_v, rows_v, gsem):
        nc = lax.axis_size("c")
        wid = lax.axis_index("s") * nc + lax.axis_index("c")
        base = wid * b_per_w

        def body(i, carry):
            off = base + i * chunk
            pltpu.sync_copy(idx_hbm.at[pl.ds(off, chunk)], idx_v)
            pltpu.async_copy(table_hbm.at[idx_v], rows_v, gsem).wait()
            pltpu.sync_copy(rows_v, out_hbm.at[pl.ds(off, chunk)])
            return carry

        lax.fori_loop(0, n_chunks_per_w, body, 0)

    return k


def kernel(ids, table):
    batch, n_fields = ids.shape
    vocab, dim = table.shape
    n = batch * n_fields
    n_workers = 32
    chunk = 832
    ids_flat = ids.reshape(n)
    out_flat = _sc_gather(n, dim, n_workers, chunk)(ids_flat, table)
    return out_flat.reshape(batch, n_fields, dim)


# speedup vs baseline: 6.3561x; 1.0117x over previous
import functools

import jax
import jax.numpy as jnp
from jax import lax
from jax.experimental import pallas as pl
from jax.experimental.pallas import tpu as pltpu
from jax.experimental.pallas import tpu_sc as plsc


def _sc_gather(n_rows, dim, n_workers, chunk):
    n_chunks_per_w = (n_rows // n_workers) // chunk
    b_per_w = n_rows // n_workers
    mesh = plsc.VectorSubcoreMesh(core_axis_name="c", subcore_axis_name="s")

    @functools.partial(
        pl.kernel,
        out_type=jax.ShapeDtypeStruct((n_rows, dim), jnp.float32),
        mesh=mesh,
        scratch_types=[
            pltpu.VMEM((2, chunk), jnp.int32),
            pltpu.VMEM((2, chunk, dim), jnp.float32),
            pltpu.SemaphoreType.DMA,
            pltpu.SemaphoreType.DMA,
            pltpu.SemaphoreType.DMA,
            pltpu.SemaphoreType.DMA,
        ],
        compiler_params=pltpu.CompilerParams(use_tc_tiling_on_sc=False),
    )
    def k(idx_hbm, table_hbm, out_hbm, idx_v, rows_v, gsem0, gsem1, wsem0, wsem1):
        nc = lax.axis_size("c")
        wid = lax.axis_index("s") * nc + lax.axis_index("c")
        base = wid * b_per_w
        gsem = (gsem0, gsem1)
        wsem = (wsem0, wsem1)

        def idx_slice(i):
            return idx_hbm.at[pl.ds(base + i * chunk, chunk)]

        def out_slice(i):
            return out_hbm.at[pl.ds(base + i * chunk, chunk)]

        gathers = [None, None]
        writes = [None, None]
        pltpu.sync_copy(idx_slice(0), idx_v.at[0])
        gathers[0] = pltpu.async_copy(table_hbm.at[idx_v.at[0]], rows_v.at[0], gsem[0])
        for i in range(n_chunks_per_w):
            b = i % 2
            nb = 1 - b
            if i + 1 < n_chunks_per_w:
                if writes[nb] is not None:
                    writes[nb].wait()
                pltpu.sync_copy(idx_slice(i + 1), idx_v.at[nb])
                gathers[nb] = pltpu.async_copy(
                    table_hbm.at[idx_v.at[nb]], rows_v.at[nb], gsem[nb]
                )
            gathers[b].wait()
            writes[b] = pltpu.async_copy(rows_v.at[b], out_slice(i), wsem[b])
        for b in range(2):
            if writes[b] is not None:
                writes[b].wait()

    return k


def kernel(ids, table):
    batch, n_fields = ids.shape
    vocab, dim = table.shape
    n = batch * n_fields
    n_workers = 32
    chunk = 832
    ids_flat = ids.reshape(n)
    out_flat = _sc_gather(n, dim, n_workers, chunk)(ids_flat, table)
    return out_flat.reshape(batch, n_fields, dim)
